# baseline (speedup 1.0000x reference)
import functools

import jax
import jax.numpy as jnp
from jax import lax
from jax.experimental import pallas as pl
from jax.experimental.pallas import tpu as pltpu
from jax.experimental.pallas import tpu_sc as plsc

HID = 64
DW = 128
XO = HID
DX = 16
NC = 2
NS = 16
NTILES = NC * NS
CHUNK = 128
BE = 1024
NB = 1000
F32 = jnp.float32


def _rup(x, m):
    return (x + m - 1) // m * m


def _silu(z):
    return z * jax.nn.sigmoid(z)


def _mm(a, b):
    return jnp.dot(a, b, preferred_element_type=F32)


def _sc_mesh():
    return plsc.VectorSubcoreMesh(core_axis_name="c", subcore_axis_name="s")


def _sc_params():
    return pltpu.CompilerParams(use_tc_tiling_on_sc=False)


def _sc_gather(table, idx2d):
    nchunks = idx2d.shape[0]
    per_tile = nchunks // NTILES
    out_rows = nchunks * CHUNK

    @functools.partial(
        pl.kernel,
        out_type=jax.ShapeDtypeStruct((out_rows, DW), F32),
        mesh=_sc_mesh(),
        scratch_types=[
            pltpu.VMEM((per_tile, CHUNK), jnp.int32),
            pltpu.VMEM((CHUNK, DW), F32),
            pltpu.VMEM((CHUNK, DW), F32),
            pltpu.VMEM((CHUNK, DW), F32),
            pltpu.VMEM((CHUNK, DW), F32),
            pltpu.SemaphoreType.DMA,
            pltpu.SemaphoreType.DMA,
        ],
    )
    def gk(table_hbm, idx_hbm, out_hbm, idx_v, b0, b1, b2, b3, gsem, wsem):
        wid = lax.axis_index("s") * NC + lax.axis_index("c")
        base = wid * per_tile
        pltpu.sync_copy(idx_hbm.at[pl.ds(base, per_tile)], idx_v)
        bufs = (b0, b1, b2, b3)

        @pl.loop(0, per_tile, step=4)
        def _(j):
            gs = [
                pltpu.async_copy(table_hbm.at[idx_v.at[j + t]], bufs[t], gsem)
                for t in range(4)
            ]
            for g in gs:
                g.wait()
            ws = [
                pltpu.async_copy(
                    bufs[t], out_hbm.at[pl.ds((base + j + t) * CHUNK, CHUNK)], wsem
                )
                for t in range(4)
            ]
            for w in ws:
                w.wait()

    return gk(table, idx2d)


def _sc_scatter(contrib, idxsc, zeros_init, acc_rows, col0, w):
    schunks = idxsc.shape[1]
    per_tile = schunks // NS
    zrows = acc_rows // NS
    ki = next(k for k in (56, 40, 24, 16, 8) if per_tile % k == 0)

    @functools.partial(
        pl.kernel,
        out_type=jax.ShapeDtypeStruct((NC, acc_rows, w), F32),
        mesh=_sc_mesh(),
        compiler_params=_sc_params(),
        scratch_types=[
            pltpu.VMEM((ki, CHUNK), jnp.int32),
            pltpu.VMEM((CHUNK, w), F32),
            pltpu.VMEM((CHUNK, w), F32),
            pltpu.VMEM_SHARED((acc_rows, w), F32),
            pltpu.SemaphoreType.DMA,
        ],
    )
    def sk(contrib_hbm, idx_hbm, zeros_hbm, out_hbm, idx_v, c0, c1, acc, lsem):
        cid = lax.axis_index("c")
        sid = lax.axis_index("s")
        pltpu.sync_copy(zeros_hbm, acc.at[pl.ds(sid * zrows, zrows)])
        plsc.subcore_barrier()

        @pl.loop(0, per_tile, step=ki)
        def _(jo):
            pltpu.sync_copy(
                idx_hbm.at[cid, pl.ds(sid * per_tile + jo, ki)], idx_v
            )

            @pl.loop(0, ki, step=2)
            def _(t):
                j = jo + t
                l0 = pltpu.async_copy(
                    contrib_hbm.at[
                        pl.ds((sid * per_tile + j) * CHUNK, CHUNK),
                        pl.ds(col0, w),
                    ],
                    c0, lsem,
                )
                l1 = pltpu.async_copy(
                    contrib_hbm.at[
                        pl.ds((sid * per_tile + j + 1) * CHUNK, CHUNK),
                        pl.ds(col0, w),
                    ],
                    c1, lsem,
                )
                l0.wait()
                pltpu.sync_copy(c0, acc.at[idx_v.at[t]], add=True)
                l1.wait()
                pltpu.sync_copy(c1, acc.at[idx_v.at[t + 1]], add=True)

        plsc.subcore_barrier()
        pltpu.sync_copy(
            acc.at[pl.ds(sid * zrows, zrows)],
            out_hbm.at[cid, pl.ds(sid * zrows, zrows)],
        )

    return sk(contrib, idxsc, zeros_init)


def _tc_init(nodes, loc, emb_W, emb_b, n):
    nblk = n // NB

    def body(nd, lc, ew, eb, out):
        h0 = nd[...] * ew[...] + eb[...]
        out[...] = jnp.concatenate(
            [h0, lc[...], jnp.zeros((NB, DW - XO - 3), F32)], axis=1
        )

    return pl.pallas_call(
        body,
        grid=(nblk,),
        in_specs=[
            pl.BlockSpec((NB, 1), lambda i: (i, 0)),
            pl.BlockSpec((NB, 3), lambda i: (i, 0)),
            pl.BlockSpec((1, HID), lambda i: (0, 0)),
            pl.BlockSpec((1, HID), lambda i: (0, 0)),
        ],
        out_specs=pl.BlockSpec((NB, DW), lambda i: (i, 0)),
        out_shape=jax.ShapeDtypeStruct((n, DW), F32),
    )(nodes, loc, emb_W.reshape(1, HID), emb_b.reshape(1, HID))


def _tc_edge(G, ea, lp, ep):
    grid = ep // BE
    col_off = ep // BE

    def body(gr, gc, ear, w1, b1, w2, b2, cw1, cb1, cw2, out):
        hr = gr[:, :HID]
        hc = gc[:, :HID]
        cd = gr[:, XO:XO + 3] - gc[:, XO:XO + 3]
        radial = jnp.sum(cd * cd, axis=1, keepdims=True)
        z = (
            _mm(hr, w1[:HID])
            + _mm(hc, w1[HID:2 * HID])
            + radial * w1[2 * HID:2 * HID + 1]
            + _mm(ear[...], w1[2 * HID + 1:])
            + b1[...]
        )
        m = _silu(z)
        m2 = _silu(_mm(m, w2[...]) + b2[...])
        cmid = _silu(_mm(m2, cw1[...]) + cb1[...])
        cm = _mm(cmid, cw2[...])
        out[...] = jnp.concatenate(
            [m2, cd * cm, jnp.ones((BE, 1), F32),
             jnp.zeros((BE, DW - HID - 4), F32)],
            axis=1,
        )

    full = lambda shape: pl.BlockSpec(shape, lambda e: tuple(0 for _ in shape))
    return pl.pallas_call(
        body,
        grid=(grid,),
        in_specs=[
            pl.BlockSpec((BE, DW), lambda e: (e, 0)),
            pl.BlockSpec((BE, DW), lambda e: (e + col_off, 0)),
            pl.BlockSpec((BE, 2), lambda e: (e, 0)),
            full((2 * HID + 3, HID)),
            full((1, HID)),
            full((HID, HID)),
            full((1, HID)),
            full((HID, HID)),
            full((1, HID)),
            full((HID, 1)),
        ],
        out_specs=pl.BlockSpec((BE, DW), lambda e: (e, 0)),
        out_shape=jax.ShapeDtypeStruct((ep, DW), F32),
    )(
        G, G, ea,
        lp["eW1"], lp["eb1"].reshape(1, HID),
        lp["eW2"], lp["eb2"].reshape(1, HID),
        lp["cW1"], lp["cb1"].reshape(1, HID),
        lp["cW2"],
    )


def _tc_node(tbl, aggm, aggx, vel, lp, n):
    nhalf = n // NC
    nblk = nhalf // NB

    def body(tb, agm, agx, ve, vw1, vb1, vw2, vb2, nw1, nb1, nw2, nb2, out):
        h = tb[:, :HID]
        x = tb[:, XO:XO + 3]
        am = agm[0]
        ax = agx[0]
        xs = ax[:, :3]
        cnt = jnp.maximum(ax[:, 3:4], 1.0)
        v = _silu(_mm(h, vw1[...]) + vb1[...])
        vv = _mm(v, vw2[...]) + vb2[...]
        xn = x + xs / cnt + vv * ve[...]
        zn = _mm(h, nw1[:HID]) + _mm(am, nw1[HID:]) + nb1[...]
        hn = h + _mm(_silu(zn), nw2[...]) + nb2[...]
        out[...] = jnp.concatenate(
            [hn, xn, jnp.zeros((NB, DW - XO - 3), F32)], axis=1
        )

    full = lambda shape: pl.BlockSpec(shape, lambda c, i: tuple(0 for _ in shape))
    return pl.pallas_call(
        body,
        grid=(NC, nblk),
        in_specs=[
            pl.BlockSpec((NB, DW), lambda c, i: (c * nblk + i, 0)),
            pl.BlockSpec((1, NB, HID), lambda c, i: (c, i, 0)),
            pl.BlockSpec((1, NB, DX), lambda c, i: (c, i, 0)),
            pl.BlockSpec((NB, 3), lambda c, i: (c * nblk + i, 0)),
            full((HID, HID)),
            full((1, HID)),
            full((HID, 1)),
            full((1, 1)),
            full((2 * HID, HID)),
            full((1, HID)),
            full((HID, HID)),
            full((1, HID)),
        ],
        out_specs=pl.BlockSpec((NB, DW), lambda c, i: (c * nblk + i, 0)),
        out_shape=jax.ShapeDtypeStruct((n, DW), F32),
    )(
        tbl, aggm, aggx, vel,
        lp["vW1"], lp["vb1"].reshape(1, HID),
        lp["vW2"], lp["vb2"].reshape(1, 1),
        lp["nW1"], lp["nb1"].reshape(1, HID),
        lp["nW2"], lp["nb2"].reshape(1, HID),
    )


NSPLIT = 2


def kernel(nodes, loc, edges, vel, edge_attr, params):
    n = nodes.shape[0]
    e = edges.shape[1]
    ep = _rup(e, CHUNK * NS * 8 * NSPLIT)
    eh = ep // NSPLIT
    nhalf = n // NC
    acc_rows = _rup(nhalf + 1, CHUNK)

    row = edges[0]
    col = edges[1]
    padi = jnp.zeros((ep - e,), jnp.int32)
    rowp = jnp.concatenate([row, padi])
    colp = jnp.concatenate([col, padi])
    valid = jnp.arange(ep, dtype=jnp.int32) < e
    trash = jnp.int32(nhalf)

    idxg, idxsc, eas = [], [], []
    for h in range(NSPLIT):
        sl = slice(h * eh, (h + 1) * eh)
        rh, ch, vh = rowp[sl], colp[sl], valid[sl]
        idxg.append(jnp.concatenate([rh, ch]).reshape(2 * eh // CHUNK, CHUNK))
        parts = []
        for c in range(NC):
            in_rng = vh & (rh >= c * nhalf) & (rh < (c + 1) * nhalf)
            parts.append(jnp.where(in_rng, rh - c * nhalf, trash))
        idxsc.append(jnp.stack(parts).reshape(NC, eh // CHUNK, CHUNK))
    ea_pad = jnp.concatenate([edge_attr, jnp.zeros((ep - e, 2), F32)])
    for h in range(NSPLIT):
        eas.append(ea_pad[h * eh:(h + 1) * eh])

    zeros_m = jnp.zeros((acc_rows // NS, HID), F32)
    zeros_x = jnp.zeros((acc_rows // NS, DX), F32)

    tbl = _tc_init(nodes, loc, params["emb_W"], params["emb_b"], n)
    for lp in params["layers"]:
        aggm = aggx = None
        for h in range(NSPLIT):
            G = _sc_gather(tbl, idxg[h])
            contrib = _tc_edge(G, eas[h], lp, eh)
            am = _sc_scatter(contrib, idxsc[h], zeros_m, acc_rows, 0, HID)
            ax = _sc_scatter(contrib, idxsc[h], zeros_x, acc_rows, HID, DX)
            aggm = am if aggm is None else aggm + am
            aggx = ax if aggx is None else aggx + ax
        tbl = _tc_node(tbl, aggm, aggx, vel, lp, n)
    return tbl[:, XO:XO + 3]


# speedup vs baseline: 1.1107x; 1.1107x over previous
import functools

import jax
import jax.numpy as jnp
from jax import lax
from jax.experimental import pallas as pl
from jax.experimental.pallas import tpu as pltpu
from jax.experimental.pallas import tpu_sc as plsc

HID = 64
HW = HID // 2
TW = 48
XO = HW
DW = 128
DX = 16
NC = 2
NS = 16
NTILES = NC * NS
CHUNK = 128
BE = 1024
NB = 1000
F32 = jnp.float32
BF16 = jnp.bfloat16


def _rup(x, m):
    return (x + m - 1) // m * m


def _silu(z):
    return z * jax.nn.sigmoid(z)


def _mm(a, b):
    return jnp.dot(a, b, preferred_element_type=F32)


def _sc_mesh():
    return plsc.VectorSubcoreMesh(core_axis_name="c", subcore_axis_name="s")


def _sc_params():
    return pltpu.CompilerParams(use_tc_tiling_on_sc=False)


def _pack_h(h):
    lo = jax.lax.bitcast_convert_type(h[:, :HW].astype(BF16), jnp.uint16)
    hi = jax.lax.bitcast_convert_type(h[:, HW:].astype(BF16), jnp.uint16)
    word = lo.astype(jnp.uint32) | (hi.astype(jnp.uint32) << 16)
    return jax.lax.bitcast_convert_type(word, F32)


def _unpack_h(w):
    wu = jax.lax.bitcast_convert_type(w, jnp.uint32)
    lo = jax.lax.bitcast_convert_type((wu & 0xFFFF).astype(jnp.uint16), BF16)
    hi = jax.lax.bitcast_convert_type((wu >> 16).astype(jnp.uint16), BF16)
    return jnp.concatenate([lo, hi], axis=1).astype(F32)


def _sc_gather(table, idx2d):
    nchunks = idx2d.shape[0]
    per_tile = nchunks // NTILES
    out_rows = nchunks * CHUNK

    @functools.partial(
        pl.kernel,
        out_type=jax.ShapeDtypeStruct((out_rows, TW), F32),
        mesh=_sc_mesh(),
        compiler_params=_sc_params(),
        scratch_types=[
            pltpu.VMEM((per_tile, CHUNK), jnp.int32),
            pltpu.VMEM((CHUNK, TW), F32),
            pltpu.VMEM((CHUNK, TW), F32),
            pltpu.VMEM((CHUNK, TW), F32),
            pltpu.VMEM((CHUNK, TW), F32),
            pltpu.SemaphoreType.DMA,
            pltpu.SemaphoreType.DMA,
        ],
    )
    def gk(table_hbm, idx_hbm, out_hbm, idx_v, b0, b1, b2, b3, gsem, wsem):
        wid = lax.axis_index("s") * NC + lax.axis_index("c")
        base = wid * per_tile
        pltpu.sync_copy(idx_hbm.at[pl.ds(base, per_tile)], idx_v)
        bufs = (b0, b1, b2, b3)

        @pl.loop(0, per_tile, step=4)
        def _(j):
            gs = [
                pltpu.async_copy(table_hbm.at[idx_v.at[j + t]], bufs[t], gsem)
                for t in range(4)
            ]
            for g in gs:
                g.wait()
            ws = [
                pltpu.async_copy(
                    bufs[t], out_hbm.at[pl.ds((base + j + t) * CHUNK, CHUNK)], wsem
                )
                for t in range(4)
            ]
            for w in ws:
                w.wait()

    return gk(table, idx2d)


def _sc_scatter(contrib, idxsc, zeros_init, acc_rows, col0, w):
    schunks = idxsc.shape[1]
    per_tile = schunks // NS
    zrows = acc_rows // NS
    ki = next(k for k in (56, 40, 24, 16, 8) if per_tile % k == 0)

    @functools.partial(
        pl.kernel,
        out_type=jax.ShapeDtypeStruct((NC, acc_rows, w), F32),
        mesh=_sc_mesh(),
        compiler_params=_sc_params(),
        scratch_types=[
            pltpu.VMEM((ki, CHUNK), jnp.int32),
            pltpu.VMEM((CHUNK, w), F32),
            pltpu.VMEM((CHUNK, w), F32),
            pltpu.VMEM_SHARED((acc_rows, w), F32),
            pltpu.SemaphoreType.DMA,
        ],
    )
    def sk(contrib_hbm, idx_hbm, zeros_hbm, out_hbm, idx_v, c0, c1, acc, lsem):
        cid = lax.axis_index("c")
        sid = lax.axis_index("s")
        pltpu.sync_copy(zeros_hbm, acc.at[pl.ds(sid * zrows, zrows)])
        plsc.subcore_barrier()

        @pl.loop(0, per_tile, step=ki)
        def _(jo):
            pltpu.sync_copy(
                idx_hbm.at[cid, pl.ds(sid * per_tile + jo, ki)], idx_v
            )

            @pl.loop(0, ki, step=2)
            def _(t):
                j = jo + t
                l0 = pltpu.async_copy(
                    contrib_hbm.at[
                        pl.ds((sid * per_tile + j) * CHUNK, CHUNK),
                        pl.ds(col0, w),
                    ],
                    c0, lsem,
                )
                l1 = pltpu.async_copy(
                    contrib_hbm.at[
                        pl.ds((sid * per_tile + j + 1) * CHUNK, CHUNK),
                        pl.ds(col0, w),
                    ],
                    c1, lsem,
                )
                l0.wait()
                pltpu.sync_copy(c0, acc.at[idx_v.at[t]], add=True)
                l1.wait()
                pltpu.sync_copy(c1, acc.at[idx_v.at[t + 1]], add=True)

        plsc.subcore_barrier()
        pltpu.sync_copy(
            acc.at[pl.ds(sid * zrows, zrows)],
            out_hbm.at[cid, pl.ds(sid * zrows, zrows)],
        )

    return sk(contrib, idxsc, zeros_init)


def _tc_init(nodes, loc, emb_W, emb_b, n):
    nblk = n // NB

    def body(nd, lc, ew, eb, out):
        h0 = nd[...] * ew[...] + eb[...]
        out[...] = jnp.concatenate(
            [_pack_h(h0), lc[...], jnp.zeros((NB, TW - XO - 3), F32)], axis=1
        )

    return pl.pallas_call(
        body,
        grid=(nblk,),
        in_specs=[
            pl.BlockSpec((NB, 1), lambda i: (i, 0)),
            pl.BlockSpec((NB, 3), lambda i: (i, 0)),
            pl.BlockSpec((1, HID), lambda i: (0, 0)),
            pl.BlockSpec((1, HID), lambda i: (0, 0)),
        ],
        out_specs=pl.BlockSpec((NB, TW), lambda i: (i, 0)),
        out_shape=jax.ShapeDtypeStruct((n, TW), F32),
    )(nodes, loc, emb_W.reshape(1, HID), emb_b.reshape(1, HID))


def _tc_edge(G, ea, lp, ep):
    grid = ep // BE
    col_off = ep // BE

    def body(gr, gc, ear, w1, b1, w2, b2, cw1, cb1, cw2, out):
        hr = _unpack_h(gr[:, :HW])
        hc = _unpack_h(gc[:, :HW])
        cd = gr[:, XO:XO + 3] - gc[:, XO:XO + 3]
        radial = jnp.sum(cd * cd, axis=1, keepdims=True)
        z = (
            _mm(hr, w1[:HID])
            + _mm(hc, w1[HID:2 * HID])
            + radial * w1[2 * HID:2 * HID + 1]
            + _mm(ear[...], w1[2 * HID + 1:])
            + b1[...]
        )
        m = _silu(z)
        m2 = _silu(_mm(m, w2[...]) + b2[...])
        cmid = _silu(_mm(m2, cw1[...]) + cb1[...])
        cm = _mm(cmid, cw2[...])
        out[...] = jnp.concatenate(
            [m2, cd * cm, jnp.ones((BE, 1), F32),
             jnp.zeros((BE, DW - HID - 4), F32)],
            axis=1,
        )

    full = lambda shape: pl.BlockSpec(shape, lambda e: tuple(0 for _ in shape))
    return pl.pallas_call(
        body,
        grid=(grid,),
        in_specs=[
            pl.BlockSpec((BE, TW), lambda e: (e, 0)),
            pl.BlockSpec((BE, TW), lambda e: (e + col_off, 0)),
            pl.BlockSpec((BE, 2), lambda e: (e, 0)),
            full((2 * HID + 3, HID)),
            full((1, HID)),
            full((HID, HID)),
            full((1, HID)),
            full((HID, HID)),
            full((1, HID)),
            full((HID, 1)),
        ],
        out_specs=pl.BlockSpec((BE, DW), lambda e: (e, 0)),
        out_shape=jax.ShapeDtypeStruct((ep, DW), F32),
    )(
        G, G, ea,
        lp["eW1"], lp["eb1"].reshape(1, HID),
        lp["eW2"], lp["eb2"].reshape(1, HID),
        lp["cW1"], lp["cb1"].reshape(1, HID),
        lp["cW2"],
    )


def _tc_node(tbl, aggm, aggx, vel, lp, n):
    nhalf = n // NC
    nblk = nhalf // NB

    def body(tb, agm, agx, ve, vw1, vb1, vw2, vb2, nw1, nb1, nw2, nb2, out):
        h = _unpack_h(tb[:, :HW])
        x = tb[:, XO:XO + 3]
        am = agm[0]
        ax = agx[0]
        xs = ax[:, :3]
        cnt = jnp.maximum(ax[:, 3:4], 1.0)
        v = _silu(_mm(h, vw1[...]) + vb1[...])
        vv = _mm(v, vw2[...]) + vb2[...]
        xn = x + xs / cnt + vv * ve[...]
        zn = _mm(h, nw1[:HID]) + _mm(am, nw1[HID:]) + nb1[...]
        hn = h + _mm(_silu(zn), nw2[...]) + nb2[...]
        out[...] = jnp.concatenate(
            [_pack_h(hn), xn, jnp.zeros((NB, TW - XO - 3), F32)], axis=1
        )

    full = lambda shape: pl.BlockSpec(shape, lambda c, i: tuple(0 for _ in shape))
    return pl.pallas_call(
        body,
        grid=(NC, nblk),
        in_specs=[
            pl.BlockSpec((NB, TW), lambda c, i: (c * nblk + i, 0)),
            pl.BlockSpec((1, NB, HID), lambda c, i: (c, i, 0)),
            pl.BlockSpec((1, NB, DX), lambda c, i: (c, i, 0)),
            pl.BlockSpec((NB, 3), lambda c, i: (c * nblk + i, 0)),
            full((HID, HID)),
            full((1, HID)),
            full((HID, 1)),
            full((1, 1)),
            full((2 * HID, HID)),
            full((1, HID)),
            full((HID, HID)),
            full((1, HID)),
        ],
        out_specs=pl.BlockSpec((NB, TW), lambda c, i: (c * nblk + i, 0)),
        out_shape=jax.ShapeDtypeStruct((n, TW), F32),
    )(
        tbl, aggm, aggx, vel,
        lp["vW1"], lp["vb1"].reshape(1, HID),
        lp["vW2"], lp["vb2"].reshape(1, 1),
        lp["nW1"], lp["nb1"].reshape(1, HID),
        lp["nW2"], lp["nb2"].reshape(1, HID),
    )


def kernel(nodes, loc, edges, vel, edge_attr, params):
    n = nodes.shape[0]
    e = edges.shape[1]
    ep = _rup(e, CHUNK * NS * 8)
    nhalf = n // NC
    acc_rows = _rup(nhalf + 1, CHUNK)

    row = edges[0]
    col = edges[1]
    padi = jnp.zeros((ep - e,), jnp.int32)
    rowp = jnp.concatenate([row, padi])
    colp = jnp.concatenate([col, padi])
    idxg = jnp.concatenate([rowp, colp]).reshape(2 * ep // CHUNK, CHUNK)

    valid = jnp.arange(ep, dtype=jnp.int32) < e
    trash = jnp.int32(nhalf)
    parts = []
    for c in range(NC):
        in_rng = valid & (rowp >= c * nhalf) & (rowp < (c + 1) * nhalf)
        parts.append(jnp.where(in_rng, rowp - c * nhalf, trash))
    idxsc = jnp.stack(parts).reshape(NC, ep // CHUNK, CHUNK)

    ea_pad = jnp.concatenate([edge_attr, jnp.zeros((ep - e, 2), F32)])
    zeros_m = jnp.zeros((acc_rows // NS, HID), F32)
    zeros_x = jnp.zeros((acc_rows // NS, DX), F32)

    tbl = _tc_init(nodes, loc, params["emb_W"], params["emb_b"], n)
    for lp in params["layers"]:
        G = _sc_gather(tbl, idxg)
        contrib = _tc_edge(G, ea_pad, lp, ep)
        aggm = _sc_scatter(contrib, idxsc, zeros_m, acc_rows, 0, HID)
        aggx = _sc_scatter(contrib, idxsc, zeros_x, acc_rows, HID, DX)
        tbl = _tc_node(tbl, aggm, aggx, vel, lp, n)
    return tbl[:, XO:XO + 3]


# speedup vs baseline: 1.1886x; 1.0701x over previous
import functools

import jax
import jax.numpy as jnp
from jax import lax
from jax.experimental import pallas as pl
from jax.experimental.pallas import tpu as pltpu
from jax.experimental.pallas import tpu_sc as plsc

HID = 64
TW = 128
XO = HID
DW = 128
DX = 16
NC = 2
NS = 16
NTILES = NC * NS
CHUNK = 128
BE = 1024
NB = 1000
F32 = jnp.float32
BF16 = jnp.bfloat16


def _rup(x, m):
    return (x + m - 1) // m * m


def _silu(z):
    return z * jax.nn.sigmoid(z)


def _mm(a, b):
    return jnp.dot(a, b, preferred_element_type=F32)


def _sc_mesh():
    return plsc.VectorSubcoreMesh(core_axis_name="c", subcore_axis_name="s")


def _sc_params():
    return pltpu.CompilerParams(use_tc_tiling_on_sc=False)


def _sc_gather(table, idx2d):
    nchunks = idx2d.shape[0]
    per_tile = nchunks // NTILES
    out_rows = nchunks * CHUNK

    @functools.partial(
        pl.kernel,
        out_type=jax.ShapeDtypeStruct((out_rows, TW), F32),
        mesh=_sc_mesh(),
        scratch_types=[
            pltpu.VMEM((per_tile, CHUNK), jnp.int32),
            pltpu.VMEM((CHUNK, TW), F32),
            pltpu.VMEM((CHUNK, TW), F32),
            pltpu.VMEM((CHUNK, TW), F32),
            pltpu.VMEM((CHUNK, TW), F32),
            pltpu.SemaphoreType.DMA,
            pltpu.SemaphoreType.DMA,
        ],
    )
    def gk(table_hbm, idx_hbm, out_hbm, idx_v, b0, b1, b2, b3, gsem, wsem):
        wid = lax.axis_index("s") * NC + lax.axis_index("c")
        base = wid * per_tile
        pltpu.sync_copy(idx_hbm.at[pl.ds(base, per_tile)], idx_v)
        bufs = (b0, b1, b2, b3)

        @pl.loop(0, per_tile, step=4)
        def _(j):
            gs = [
                pltpu.async_copy(table_hbm.at[idx_v.at[j + t]], bufs[t], gsem)
                for t in range(4)
            ]
            for g in gs:
                g.wait()
            ws = [
                pltpu.async_copy(
                    bufs[t], out_hbm.at[pl.ds((base + j + t) * CHUNK, CHUNK)], wsem
                )
                for t in range(4)
            ]
            for w in ws:
                w.wait()

    return gk(table, idx2d)


def _sc_scatter(contrib, idxsc, zeros_init, acc_rows, col0, w):
    schunks = idxsc.shape[1]
    per_tile = schunks // NS
    zrows = acc_rows // NS
    ki = next(k for k in (56, 40, 24, 16, 8) if per_tile % k == 0)

    @functools.partial(
        pl.kernel,
        out_type=jax.ShapeDtypeStruct((NC, acc_rows, w), F32),
        mesh=_sc_mesh(),
        compiler_params=_sc_params(),
        scratch_types=[
            pltpu.VMEM((ki, CHUNK), jnp.int32),
            pltpu.VMEM((CHUNK, w), F32),
            pltpu.VMEM((CHUNK, w), F32),
            pltpu.VMEM_SHARED((acc_rows, w), F32),
            pltpu.SemaphoreType.DMA,
            pltpu.SemaphoreType.DMA,
        ],
    )
    def sk(contrib_hbm, idx_hbm, zeros_hbm, out_hbm, idx_v, c0, c1, acc, lsem, asem):
        cid = lax.axis_index("c")
        sid = lax.axis_index("s")
        pltpu.sync_copy(zeros_hbm, acc.at[pl.ds(sid * zrows, zrows)])
        plsc.subcore_barrier()

        @pl.loop(0, per_tile, step=ki)
        def _(jo):
            pltpu.sync_copy(
                idx_hbm.at[cid, pl.ds(sid * per_tile + jo, ki)], idx_v
            )

            @pl.loop(0, ki, step=2)
            def _(t):
                j = jo + t
                l0 = pltpu.async_copy(
                    contrib_hbm.at[
                        pl.ds((sid * per_tile + j) * CHUNK, CHUNK),
                        pl.ds(col0, w),
                    ],
                    c0, lsem,
                )
                l1 = pltpu.async_copy(
                    contrib_hbm.at[
                        pl.ds((sid * per_tile + j + 1) * CHUNK, CHUNK),
                        pl.ds(col0, w),
                    ],
                    c1, lsem,
                )
                l0.wait()
                a0 = pltpu.async_copy(c0, acc.at[idx_v.at[t]], asem, add=True)
                l1.wait()
                a1 = pltpu.async_copy(c1, acc.at[idx_v.at[t + 1]], asem, add=True)
                a0.wait()
                a1.wait()

        plsc.subcore_barrier()
        pltpu.sync_copy(
            acc.at[pl.ds(sid * zrows, zrows)],
            out_hbm.at[cid, pl.ds(sid * zrows, zrows)],
        )

    return sk(contrib, idxsc, zeros_init)


def _tc_init(nodes, loc, emb_W, emb_b, n):
    nblk = n // NB

    def body(nd, lc, ew, eb, out):
        h0 = nd[...] * ew[...] + eb[...]
        out[...] = jnp.concatenate(
            [h0, lc[...], jnp.zeros((NB, TW - XO - 3), F32)], axis=1
        )

    return pl.pallas_call(
        body,
        grid=(nblk,),
        in_specs=[
            pl.BlockSpec((NB, 1), lambda i: (i, 0)),
            pl.BlockSpec((NB, 3), lambda i: (i, 0)),
            pl.BlockSpec((1, HID), lambda i: (0, 0)),
            pl.BlockSpec((1, HID), lambda i: (0, 0)),
        ],
        out_specs=pl.BlockSpec((NB, TW), lambda i: (i, 0)),
        out_shape=jax.ShapeDtypeStruct((n, TW), F32),
    )(nodes, loc, emb_W.reshape(1, HID), emb_b.reshape(1, HID))


def _tc_edge(G, ea, lp, ep):
    grid = ep // BE
    col_off = ep // BE

    def body(gr, gc, ear, w1, b1, w2, b2, cw1, cb1, cw2, out):
        hr = gr[:, :HID]
        hc = gc[:, :HID]
        cd = gr[:, XO:XO + 3] - gc[:, XO:XO + 3]
        radial = jnp.sum(cd * cd, axis=1, keepdims=True)
        z = (
            _mm(hr, w1[:HID])
            + _mm(hc, w1[HID:2 * HID])
            + radial * w1[2 * HID:2 * HID + 1]
            + _mm(ear[...], w1[2 * HID + 1:])
            + b1[...]
        )
        m = _silu(z)
        m2 = _silu(_mm(m, w2[...]) + b2[...])
        cmid = _silu(_mm(m2, cw1[...]) + cb1[...])
        cm = _mm(cmid, cw2[...])
        out[...] = jnp.concatenate(
            [m2, cd * cm, jnp.ones((BE, 1), F32),
             jnp.zeros((BE, DW - HID - 4), F32)],
            axis=1,
        )

    full = lambda shape: pl.BlockSpec(shape, lambda e: tuple(0 for _ in shape))
    return pl.pallas_call(
        body,
        grid=(grid,),
        in_specs=[
            pl.BlockSpec((BE, TW), lambda e: (e, 0)),
            pl.BlockSpec((BE, TW), lambda e: (e + col_off, 0)),
            pl.BlockSpec((BE, 2), lambda e: (e, 0)),
            full((2 * HID + 3, HID)),
            full((1, HID)),
            full((HID, HID)),
            full((1, HID)),
            full((HID, HID)),
            full((1, HID)),
            full((HID, 1)),
        ],
        out_specs=pl.BlockSpec((BE, DW), lambda e: (e, 0)),
        out_shape=jax.ShapeDtypeStruct((ep, DW), F32),
    )(
        G, G, ea,
        lp["eW1"], lp["eb1"].reshape(1, HID),
        lp["eW2"], lp["eb2"].reshape(1, HID),
        lp["cW1"], lp["cb1"].reshape(1, HID),
        lp["cW2"],
    )


def _tc_node(tbl, aggm, aggx, vel, lp, n):
    nhalf = n // NC
    nblk = nhalf // NB

    def body(tb, agm, agx, ve, vw1, vb1, vw2, vb2, nw1, nb1, nw2, nb2, out):
        h = tb[:, :HID]
        x = tb[:, XO:XO + 3]
        am = agm[0]
        xs = agx[0][:, :3]
        cnt = jnp.maximum(agx[0][:, 3:4], 1.0)
        v = _silu(_mm(h, vw1[...]) + vb1[...])
        vv = _mm(v, vw2[...]) + vb2[...]
        xn = x + xs / cnt + vv * ve[...]
        zn = _mm(h, nw1[:HID]) + _mm(am, nw1[HID:]) + nb1[...]
        hn = h + _mm(_silu(zn), nw2[...]) + nb2[...]
        out[...] = jnp.concatenate(
            [hn, xn, jnp.zeros((NB, TW - XO - 3), F32)], axis=1
        )

    full = lambda shape: pl.BlockSpec(shape, lambda c, i: tuple(0 for _ in shape))
    return pl.pallas_call(
        body,
        grid=(NC, nblk),
        in_specs=[
            pl.BlockSpec((NB, TW), lambda c, i: (c * nblk + i, 0)),
            pl.BlockSpec((1, NB, HID), lambda c, i: (c, i, 0)),
            pl.BlockSpec((1, NB, DX), lambda c, i: (c, i, 0)),
            pl.BlockSpec((NB, 3), lambda c, i: (c * nblk + i, 0)),
            full((HID, HID)),
            full((1, HID)),
            full((HID, 1)),
            full((1, 1)),
            full((2 * HID, HID)),
            full((1, HID)),
            full((HID, HID)),
            full((1, HID)),
        ],
        out_specs=pl.BlockSpec((NB, TW), lambda c, i: (c * nblk + i, 0)),
        out_shape=jax.ShapeDtypeStruct((n, TW), F32),
    )(
        tbl, aggm, aggx, vel,
        lp["vW1"], lp["vb1"].reshape(1, HID),
        lp["vW2"], lp["vb2"].reshape(1, 1),
        lp["nW1"], lp["nb1"].reshape(1, HID),
        lp["nW2"], lp["nb2"].reshape(1, HID),
    )


def kernel(nodes, loc, edges, vel, edge_attr, params):
    n = nodes.shape[0]
    e = edges.shape[1]
    ep = _rup(e, CHUNK * NS * 8)
    nhalf = n // NC
    acc_rows = _rup(nhalf + 1, CHUNK)

    row = edges[0]
    col = edges[1]
    padi = jnp.zeros((ep - e,), jnp.int32)
    rowp = jnp.concatenate([row, padi])
    colp = jnp.concatenate([col, padi])
    idxg = jnp.concatenate([rowp, colp]).reshape(2 * ep // CHUNK, CHUNK)

    valid = jnp.arange(ep, dtype=jnp.int32) < e
    trash = jnp.int32(nhalf)
    parts = []
    for c in range(NC):
        in_rng = valid & (rowp >= c * nhalf) & (rowp < (c + 1) * nhalf)
        parts.append(jnp.where(in_rng, rowp - c * nhalf, trash))
    idxsc = jnp.stack(parts).reshape(NC, ep // CHUNK, CHUNK)

    ea_pad = jnp.concatenate([edge_attr, jnp.zeros((ep - e, 2), F32)])
    zeros_m = jnp.zeros((acc_rows // NS, HID), F32)
    zeros_x = jnp.zeros((acc_rows // NS, DX), F32)

    tbl = _tc_init(nodes, loc, params["emb_W"], params["emb_b"], n)
    for lp in params["layers"]:
        G = _sc_gather(tbl, idxg)
        contrib = _tc_edge(G, ea_pad, lp, ep)
        aggm = _sc_scatter(contrib, idxsc, zeros_m, acc_rows, 0, HID)
        aggx = _sc_scatter(contrib, idxsc, zeros_x, acc_rows, HID, DX)
        tbl = _tc_node(tbl, aggm, aggx, vel, lp, n)
    return tbl[:, XO:XO + 3]


# speedup vs baseline: 1.2069x; 1.0154x over previous
import functools

import jax
import jax.numpy as jnp
from jax import lax
from jax.experimental import pallas as pl
from jax.experimental.pallas import tpu as pltpu
from jax.experimental.pallas import tpu_sc as plsc

HID = 64
TW = 128
XO = HID
DW = 128
DX = 16
NC = 2
NS = 16
NTILES = NC * NS
CHUNK = 128
BE = 1024
NB = 1000
F32 = jnp.float32
BF16 = jnp.bfloat16


def _rup(x, m):
    return (x + m - 1) // m * m


def _silu(z):
    return z * jax.nn.sigmoid(z)


def _mm(a, b):
    return jnp.dot(a, b, preferred_element_type=F32)


def _sc_mesh():
    return plsc.VectorSubcoreMesh(core_axis_name="c", subcore_axis_name="s")


def _sc_params():
    return pltpu.CompilerParams(use_tc_tiling_on_sc=False)


def _sc_gather(table, idx2d):
    nchunks = idx2d.shape[0]
    per_tile = nchunks // NTILES
    out_rows = nchunks * CHUNK

    gi = next(k for k in (56, 40, 24, 16, 8) if per_tile % k == 0)
    nd = next(k for k in (7, 5, 4, 2) if gi % k == 0)

    @functools.partial(
        pl.kernel,
        out_type=jax.ShapeDtypeStruct((out_rows, TW), F32),
        mesh=_sc_mesh(),
        scratch_types=[
            pltpu.VMEM((gi, CHUNK), jnp.int32),
        ]
        + [pltpu.VMEM((CHUNK, TW), F32) for _ in range(7)]
        + [
            pltpu.SemaphoreType.DMA,
            pltpu.SemaphoreType.DMA,
        ],
    )
    def gk(table_hbm, idx_hbm, out_hbm, idx_v, b0, b1, b2, b3, b4, b5, b6,
           gsem, wsem):
        wid = lax.axis_index("s") * NC + lax.axis_index("c")
        base = wid * per_tile
        bufs = (b0, b1, b2, b3, b4, b5, b6)

        @pl.loop(0, per_tile, step=gi)
        def _(jo):
            pltpu.sync_copy(idx_hbm.at[pl.ds(base + jo, gi)], idx_v)

            @pl.loop(0, gi, step=nd)
            def _(t):
                j = jo + t
                gs = [
                    pltpu.async_copy(table_hbm.at[idx_v.at[t + k]], bufs[k], gsem)
                    for k in range(nd)
                ]
                ws = []
                for k in range(nd):
                    gs[k].wait()
                    ws.append(pltpu.async_copy(
                        bufs[k],
                        out_hbm.at[pl.ds((base + j + k) * CHUNK, CHUNK)],
                        wsem,
                    ))
                for w in ws:
                    w.wait()

    return gk(table, idx2d)


def _sc_scatter(contrib, idxsc, zeros_init, acc_rows, col0, w):
    schunks = idxsc.shape[1]
    per_tile = schunks // NS
    zrows = acc_rows // NS
    ki = next(k for k in (56, 40, 24, 16, 8) if per_tile % k == 0)

    @functools.partial(
        pl.kernel,
        out_type=jax.ShapeDtypeStruct((NC, acc_rows, w), F32),
        mesh=_sc_mesh(),
        compiler_params=_sc_params(),
        scratch_types=[
            pltpu.VMEM((ki, CHUNK), jnp.int32),
            pltpu.VMEM((CHUNK, w), F32),
            pltpu.VMEM((CHUNK, w), F32),
            pltpu.VMEM_SHARED((acc_rows, w), F32),
            pltpu.SemaphoreType.DMA,
            pltpu.SemaphoreType.DMA,
        ],
    )
    def sk(contrib_hbm, idx_hbm, zeros_hbm, out_hbm, idx_v, c0, c1, acc, lsem, asem):
        cid = lax.axis_index("c")
        sid = lax.axis_index("s")
        pltpu.sync_copy(zeros_hbm, acc.at[pl.ds(sid * zrows, zrows)])
        plsc.subcore_barrier()

        @pl.loop(0, per_tile, step=ki)
        def _(jo):
            pltpu.sync_copy(
                idx_hbm.at[cid, pl.ds(sid * per_tile + jo, ki)], idx_v
            )

            @pl.loop(0, ki, step=2)
            def _(t):
                j = jo + t
                l0 = pltpu.async_copy(
                    contrib_hbm.at[
                        pl.ds((sid * per_tile + j) * CHUNK, CHUNK),
                        pl.ds(col0, w),
                    ],
                    c0, lsem,
                )
                l1 = pltpu.async_copy(
                    contrib_hbm.at[
                        pl.ds((sid * per_tile + j + 1) * CHUNK, CHUNK),
                        pl.ds(col0, w),
                    ],
                    c1, lsem,
                )
                l0.wait()
                a0 = pltpu.async_copy(c0, acc.at[idx_v.at[t]], asem, add=True)
                l1.wait()
                a1 = pltpu.async_copy(c1, acc.at[idx_v.at[t + 1]], asem, add=True)
                a0.wait()
                a1.wait()

        plsc.subcore_barrier()
        pltpu.sync_copy(
            acc.at[pl.ds(sid * zrows, zrows)],
            out_hbm.at[cid, pl.ds(sid * zrows, zrows)],
        )

    return sk(contrib, idxsc, zeros_init)


def _tc_init(nodes, loc, emb_W, emb_b, n):
    nblk = n // NB

    def body(nd, lc, ew, eb, out):
        h0 = nd[...] * ew[...] + eb[...]
        out[...] = jnp.concatenate(
            [h0, lc[...], jnp.zeros((NB, TW - XO - 3), F32)], axis=1
        )

    return pl.pallas_call(
        body,
        grid=(nblk,),
        in_specs=[
            pl.BlockSpec((NB, 1), lambda i: (i, 0)),
            pl.BlockSpec((NB, 3), lambda i: (i, 0)),
            pl.BlockSpec((1, HID), lambda i: (0, 0)),
            pl.BlockSpec((1, HID), lambda i: (0, 0)),
        ],
        out_specs=pl.BlockSpec((NB, TW), lambda i: (i, 0)),
        out_shape=jax.ShapeDtypeStruct((n, TW), F32),
    )(nodes, loc, emb_W.reshape(1, HID), emb_b.reshape(1, HID))


def _tc_edge(G, ea, lp, ep):
    grid = ep // BE
    col_off = ep // BE

    def body(gr, gc, ear, w1, b1, w2, b2, cw1, cb1, cw2, out):
        hr = gr[:, :HID]
        hc = gc[:, :HID]
        cd = gr[:, XO:XO + 3] - gc[:, XO:XO + 3]
        radial = jnp.sum(cd * cd, axis=1, keepdims=True)
        z = (
            _mm(hr, w1[:HID])
            + _mm(hc, w1[HID:2 * HID])
            + radial * w1[2 * HID:2 * HID + 1]
            + _mm(ear[...], w1[2 * HID + 1:])
            + b1[...]
        )
        m = _silu(z)
        m2 = _silu(_mm(m, w2[...]) + b2[...])
        cmid = _silu(_mm(m2, cw1[...]) + cb1[...])
        cm = _mm(cmid, cw2[...])
        out[...] = jnp.concatenate(
            [m2, cd * cm, jnp.ones((BE, 1), F32),
             jnp.zeros((BE, DW - HID - 4), F32)],
            axis=1,
        )

    full = lambda shape: pl.BlockSpec(shape, lambda e: tuple(0 for _ in shape))
    return pl.pallas_call(
        body,
        grid=(grid,),
        in_specs=[
            pl.BlockSpec((BE, TW), lambda e: (e, 0)),
            pl.BlockSpec((BE, TW), lambda e: (e + col_off, 0)),
            pl.BlockSpec((BE, 2), lambda e: (e, 0)),
            full((2 * HID + 3, HID)),
            full((1, HID)),
            full((HID, HID)),
            full((1, HID)),
            full((HID, HID)),
            full((1, HID)),
            full((HID, 1)),
        ],
        out_specs=pl.BlockSpec((BE, DW), lambda e: (e, 0)),
        out_shape=jax.ShapeDtypeStruct((ep, DW), F32),
    )(
        G, G, ea,
        lp["eW1"], lp["eb1"].reshape(1, HID),
        lp["eW2"], lp["eb2"].reshape(1, HID),
        lp["cW1"], lp["cb1"].reshape(1, HID),
        lp["cW2"],
    )


def _tc_node(tbl, aggm, aggx, vel, lp, n):
    nhalf = n // NC
    nblk = nhalf // NB

    def body(tb, agm, agx, ve, vw1, vb1, vw2, vb2, nw1, nb1, nw2, nb2, out):
        h = tb[:, :HID]
        x = tb[:, XO:XO + 3]
        am = agm[0]
        xs = agx[0][:, :3]
        cnt = jnp.maximum(agx[0][:, 3:4], 1.0)
        v = _silu(_mm(h, vw1[...]) + vb1[...])
        vv = _mm(v, vw2[...]) + vb2[...]
        xn = x + xs / cnt + vv * ve[...]
        zn = _mm(h, nw1[:HID]) + _mm(am, nw1[HID:]) + nb1[...]
        hn = h + _mm(_silu(zn), nw2[...]) + nb2[...]
        out[...] = jnp.concatenate(
            [hn, xn, jnp.zeros((NB, TW - XO - 3), F32)], axis=1
        )

    full = lambda shape: pl.BlockSpec(shape, lambda c, i: tuple(0 for _ in shape))
    return pl.pallas_call(
        body,
        grid=(NC, nblk),
        in_specs=[
            pl.BlockSpec((NB, TW), lambda c, i: (c * nblk + i, 0)),
            pl.BlockSpec((1, NB, HID), lambda c, i: (c, i, 0)),
            pl.BlockSpec((1, NB, DX), lambda c, i: (c, i, 0)),
            pl.BlockSpec((NB, 3), lambda c, i: (c * nblk + i, 0)),
            full((HID, HID)),
            full((1, HID)),
            full((HID, 1)),
            full((1, 1)),
            full((2 * HID, HID)),
            full((1, HID)),
            full((HID, HID)),
            full((1, HID)),
        ],
        out_specs=pl.BlockSpec((NB, TW), lambda c, i: (c * nblk + i, 0)),
        out_shape=jax.ShapeDtypeStruct((n, TW), F32),
    )(
        tbl, aggm, aggx, vel,
        lp["vW1"], lp["vb1"].reshape(1, HID),
        lp["vW2"], lp["vb2"].reshape(1, 1),
        lp["nW1"], lp["nb1"].reshape(1, HID),
        lp["nW2"], lp["nb2"].reshape(1, HID),
    )


def kernel(nodes, loc, edges, vel, edge_attr, params):
    n = nodes.shape[0]
    e = edges.shape[1]
    ep = _rup(e, CHUNK * NS * 8)
    nhalf = n // NC
    acc_rows = _rup(nhalf + 1, CHUNK)

    row = edges[0]
    col = edges[1]
    padi = jnp.zeros((ep - e,), jnp.int32)
    rowp = jnp.concatenate([row, padi])
    colp = jnp.concatenate([col, padi])
    idxg = jnp.concatenate([rowp, colp]).reshape(2 * ep // CHUNK, CHUNK)

    valid = jnp.arange(ep, dtype=jnp.int32) < e
    trash = jnp.int32(nhalf)
    parts = []
    for c in range(NC):
        in_rng = valid & (rowp >= c * nhalf) & (rowp < (c + 1) * nhalf)
        parts.append(jnp.where(in_rng, rowp - c * nhalf, trash))
    idxsc = jnp.stack(parts).reshape(NC, ep // CHUNK, CHUNK)

    ea_pad = jnp.concatenate([edge_attr, jnp.zeros((ep - e, 2), F32)])
    zeros_m = jnp.zeros((acc_rows // NS, HID), F32)
    zeros_x = jnp.zeros((acc_rows // NS, DX), F32)

    tbl = _tc_init(nodes, loc, params["emb_W"], params["emb_b"], n)
    for lp in params["layers"]:
        G = _sc_gather(tbl, idxg)
        contrib = _tc_edge(G, ea_pad, lp, ep)
        aggm = _sc_scatter(contrib, idxsc, zeros_m, acc_rows, 0, HID)
        aggx = _sc_scatter(contrib, idxsc, zeros_x, acc_rows, HID, DX)
        tbl = _tc_node(tbl, aggm, aggx, vel, lp, n)
    return tbl[:, XO:XO + 3]


# speedup vs baseline: 1.2650x; 1.0482x over previous
import functools

import jax
import jax.numpy as jnp
from jax import lax
from jax.experimental import pallas as pl
from jax.experimental.pallas import tpu as pltpu
from jax.experimental.pallas import tpu_sc as plsc

HID = 64
TW = 128
XO = HID
DW = 128
DX = 16
NC = 2
NS = 16
NTILES = NC * NS
CHUNK = 128
BE = 1024
NB = 1000
F32 = jnp.float32
BF16 = jnp.bfloat16


def _rup(x, m):
    return (x + m - 1) // m * m


def _silu(z):
    return z * jax.nn.sigmoid(z)


def _mm(a, b):
    return jnp.dot(a, b, preferred_element_type=F32)


def _sc_mesh():
    return plsc.VectorSubcoreMesh(core_axis_name="c", subcore_axis_name="s")


def _sc_params():
    return pltpu.CompilerParams(use_tc_tiling_on_sc=False)


def _sc_gather(table, idx2d):
    nchunks = idx2d.shape[0]
    per_tile = nchunks // NTILES
    out_rows = nchunks * CHUNK

    gi = next(k for k in (56, 40, 24, 16, 8) if per_tile % k == 0)
    nd = next(k for k in (7, 5, 4, 2) if gi % k == 0)

    @functools.partial(
        pl.kernel,
        out_type=jax.ShapeDtypeStruct((out_rows, TW), F32),
        mesh=_sc_mesh(),
        scratch_types=[
            pltpu.VMEM((gi, CHUNK), jnp.int32),
        ]
        + [pltpu.VMEM((CHUNK, TW), F32) for _ in range(7)]
        + [
            pltpu.SemaphoreType.DMA,
            pltpu.SemaphoreType.DMA,
        ],
    )
    def gk(table_hbm, idx_hbm, out_hbm, idx_v, b0, b1, b2, b3, b4, b5, b6,
           gsem, wsem):
        wid = lax.axis_index("s") * NC + lax.axis_index("c")
        base = wid * per_tile
        bufs = (b0, b1, b2, b3, b4, b5, b6)

        @pl.loop(0, per_tile, step=gi)
        def _(jo):
            pltpu.sync_copy(idx_hbm.at[pl.ds(base + jo, gi)], idx_v)

            @pl.loop(0, gi, step=nd)
            def _(t):
                j = jo + t
                gs = [
                    pltpu.async_copy(table_hbm.at[idx_v.at[t + k]], bufs[k], gsem)
                    for k in range(nd)
                ]
                ws = []
                for k in range(nd):
                    gs[k].wait()
                    ws.append(pltpu.async_copy(
                        bufs[k],
                        out_hbm.at[pl.ds((base + j + k) * CHUNK, CHUNK)],
                        wsem,
                    ))
                for w in ws:
                    w.wait()

    return gk(table, idx2d)


def _sc_scatter(contrib, idxsc, zeros_init, acc_rows, col0, w):
    schunks = idxsc.shape[1]
    per_tile = schunks // NS
    zrows = acc_rows // NS
    ki = next(k for k in (56, 40, 24, 16, 8) if per_tile % k == 0)

    @functools.partial(
        pl.kernel,
        out_type=jax.ShapeDtypeStruct((NC, acc_rows, w), F32),
        mesh=_sc_mesh(),
        compiler_params=_sc_params(),
        scratch_types=[
            pltpu.VMEM((ki, CHUNK), jnp.int32),
            pltpu.VMEM((CHUNK, w), F32),
            pltpu.VMEM((CHUNK, w), F32),
            pltpu.VMEM_SHARED((acc_rows, w), F32),
            pltpu.SemaphoreType.DMA,
            pltpu.SemaphoreType.DMA,
        ],
    )
    def sk(contrib_hbm, idx_hbm, zeros_hbm, out_hbm, idx_v, c0, c1, acc, lsem, asem):
        cid = lax.axis_index("c")
        sid = lax.axis_index("s")
        pltpu.sync_copy(zeros_hbm, acc.at[pl.ds(sid * zrows, zrows)])
        plsc.subcore_barrier()

        @pl.loop(0, per_tile, step=ki)
        def _(jo):
            pltpu.sync_copy(
                idx_hbm.at[cid, pl.ds(sid * per_tile + jo, ki)], idx_v
            )

            @pl.loop(0, ki, step=2)
            def _(t):
                j = jo + t
                l0 = pltpu.async_copy(
                    contrib_hbm.at[
                        pl.ds((sid * per_tile + j) * CHUNK, CHUNK),
                        pl.ds(col0, w),
                    ],
                    c0, lsem,
                )
                l1 = pltpu.async_copy(
                    contrib_hbm.at[
                        pl.ds((sid * per_tile + j + 1) * CHUNK, CHUNK),
                        pl.ds(col0, w),
                    ],
                    c1, lsem,
                )
                l0.wait()
                a0 = pltpu.async_copy(c0, acc.at[idx_v.at[t]], asem, add=True)
                l1.wait()
                a1 = pltpu.async_copy(c1, acc.at[idx_v.at[t + 1]], asem, add=True)
                a0.wait()
                a1.wait()

        plsc.subcore_barrier()
        pltpu.sync_copy(
            acc.at[pl.ds(sid * zrows, zrows)],
            out_hbm.at[cid, pl.ds(sid * zrows, zrows)],
        )

    return sk(contrib, idxsc, zeros_init)


def _tc_init(nodes, loc, emb_W, emb_b, n):
    nblk = n // NB

    def body(nd, lc, ew, eb, out):
        h0 = nd[...] * ew[...] + eb[...]
        out[...] = jnp.concatenate(
            [h0, lc[...], jnp.zeros((NB, TW - XO - 3), F32)], axis=1
        )

    return pl.pallas_call(
        body,
        grid=(nblk,),
        in_specs=[
            pl.BlockSpec((NB, 1), lambda i: (i, 0)),
            pl.BlockSpec((NB, 3), lambda i: (i, 0)),
            pl.BlockSpec((1, HID), lambda i: (0, 0)),
            pl.BlockSpec((1, HID), lambda i: (0, 0)),
        ],
        out_specs=pl.BlockSpec((NB, TW), lambda i: (i, 0)),
        out_shape=jax.ShapeDtypeStruct((n, TW), F32),
    )(nodes, loc, emb_W.reshape(1, HID), emb_b.reshape(1, HID))


def _tc_edge(G, ea, lp, ep):
    grid = ep // BE
    col_off = ep // BE

    def body(gr, gc, ear, w1, b1, w2, b2, cw1, cb1, cw2, out):
        hr = gr[:, :HID]
        hc = gc[:, :HID]
        cd = gr[:, XO:XO + 3] - gc[:, XO:XO + 3]
        radial = jnp.sum(cd * cd, axis=1, keepdims=True)
        z = (
            _mm(hr, w1[:HID])
            + _mm(hc, w1[HID:2 * HID])
            + radial * w1[2 * HID:2 * HID + 1]
            + jax.lax.dot_general(
                ear[...], w1[2 * HID + 1:],
                (((0,), (0,)), ((), ())), preferred_element_type=F32)
            + b1[...]
        )
        m = _silu(z)
        m2 = _silu(_mm(m, w2[...]) + b2[...])
        cmid = _silu(_mm(m2, cw1[...]) + cb1[...])
        cm = _mm(cmid, cw2[...])
        out[...] = jnp.concatenate(
            [m2, cd * cm, jnp.ones((BE, 1), F32),
             jnp.zeros((BE, DW - HID - 4), F32)],
            axis=1,
        )

    full = lambda shape: pl.BlockSpec(shape, lambda e: tuple(0 for _ in shape))
    return pl.pallas_call(
        body,
        grid=(grid,),
        in_specs=[
            pl.BlockSpec((BE, TW), lambda e: (e, 0)),
            pl.BlockSpec((BE, TW), lambda e: (e + col_off, 0)),
            pl.BlockSpec((2, BE), lambda e: (0, e)),
            full((2 * HID + 3, HID)),
            full((1, HID)),
            full((HID, HID)),
            full((1, HID)),
            full((HID, HID)),
            full((1, HID)),
            full((HID, 1)),
        ],
        out_specs=pl.BlockSpec((BE, DW), lambda e: (e, 0)),
        out_shape=jax.ShapeDtypeStruct((ep, DW), F32),
    )(
        G, G, ea,
        lp["eW1"], lp["eb1"].reshape(1, HID),
        lp["eW2"], lp["eb2"].reshape(1, HID),
        lp["cW1"], lp["cb1"].reshape(1, HID),
        lp["cW2"],
    )


def _tc_node(tbl, aggm, aggx, vel, lp, n):
    nhalf = n // NC
    nblk = nhalf // NB

    def body(tb, agm, agx, ve, vw1, vb1, vw2, vb2, nw1, nb1, nw2, nb2, out):
        h = tb[:, :HID]
        x = tb[:, XO:XO + 3]
        am = agm[0]
        xs = agx[0][:, :3]
        cnt = jnp.maximum(agx[0][:, 3:4], 1.0)
        v = _silu(_mm(h, vw1[...]) + vb1[...])
        vv = _mm(v, vw2[...]) + vb2[...]
        xn = x + xs / cnt + vv * ve[...]
        zn = _mm(h, nw1[:HID]) + _mm(am, nw1[HID:]) + nb1[...]
        hn = h + _mm(_silu(zn), nw2[...]) + nb2[...]
        out[...] = jnp.concatenate(
            [hn, xn, jnp.zeros((NB, TW - XO - 3), F32)], axis=1
        )

    full = lambda shape: pl.BlockSpec(shape, lambda c, i: tuple(0 for _ in shape))
    return pl.pallas_call(
        body,
        grid=(NC, nblk),
        in_specs=[
            pl.BlockSpec((NB, TW), lambda c, i: (c * nblk + i, 0)),
            pl.BlockSpec((1, NB, HID), lambda c, i: (c, i, 0)),
            pl.BlockSpec((1, NB, DX), lambda c, i: (c, i, 0)),
            pl.BlockSpec((NB, 3), lambda c, i: (c * nblk + i, 0)),
            full((HID, HID)),
            full((1, HID)),
            full((HID, 1)),
            full((1, 1)),
            full((2 * HID, HID)),
            full((1, HID)),
            full((HID, HID)),
            full((1, HID)),
        ],
        out_specs=pl.BlockSpec((NB, TW), lambda c, i: (c * nblk + i, 0)),
        out_shape=jax.ShapeDtypeStruct((n, TW), F32),
    )(
        tbl, aggm, aggx, vel,
        lp["vW1"], lp["vb1"].reshape(1, HID),
        lp["vW2"], lp["vb2"].reshape(1, 1),
        lp["nW1"], lp["nb1"].reshape(1, HID),
        lp["nW2"], lp["nb2"].reshape(1, HID),
    )


def kernel(nodes, loc, edges, vel, edge_attr, params):
    n = nodes.shape[0]
    e = edges.shape[1]
    ep = _rup(e, CHUNK * NS * 8)
    nhalf = n // NC
    acc_rows = _rup(nhalf + 1, CHUNK)

    row = edges[0]
    col = edges[1]
    padi = jnp.zeros((ep - e,), jnp.int32)
    rowp = jnp.concatenate([row, padi])
    colp = jnp.concatenate([col, padi])
    idxg = jnp.concatenate([rowp, colp]).reshape(2 * ep // CHUNK, CHUNK)

    valid = jnp.arange(ep, dtype=jnp.int32) < e
    trash = jnp.int32(nhalf)
    parts = []
    for c in range(NC):
        in_rng = valid & (rowp >= c * nhalf) & (rowp < (c + 1) * nhalf)
        parts.append(jnp.where(in_rng, rowp - c * nhalf, trash))
    idxsc = jnp.stack(parts).reshape(NC, ep // CHUNK, CHUNK)

    ea_pad = jnp.concatenate([edge_attr.T, jnp.zeros((2, ep - e), F32)], axis=1)
    zeros_m = jnp.zeros((acc_rows // NS, HID), F32)
    zeros_x = jnp.zeros((acc_rows // NS, DX), F32)

    tbl = _tc_init(nodes, loc, params["emb_W"], params["emb_b"], n)
    for lp in params["layers"]:
        G = _sc_gather(tbl, idxg)
        contrib = _tc_edge(G, ea_pad, lp, ep)
        aggm = _sc_scatter(contrib, idxsc, zeros_m, acc_rows, 0, HID)
        aggx = _sc_scatter(contrib, idxsc, zeros_x, acc_rows, HID, DX)
        tbl = _tc_node(tbl, aggm, aggx, vel, lp, n)
    return tbl[:, XO:XO + 3]
